# Initial kernel scaffold; baseline (speedup 1.0000x reference)
import jax
import jax.numpy as jnp
from jax.experimental import pallas as pl


def kernel(inputs, relation_type, parent_table, brother_table):
    raise NotImplementedError("write your pallas kernel here")


# speedup vs baseline: 4.4021x; 4.4021x over previous
import functools

import jax
import jax.numpy as jnp
from jax import lax
from jax.experimental import pallas as pl
from jax.experimental.pallas import tpu as pltpu
from jax.experimental.pallas import tpu_sc as plsc

D_MODEL = 64
VOCAB = 66
NJ = 200
LANES = 16
CHUNK_OFFS = tuple(range(0, NJ - LANES + 1, LANES)) + (NJ - LANES,)


@functools.lru_cache(maxsize=None)
def _make_gather(nb: int, ni: int):
    info = plsc.get_sparse_core_info()
    nc, ns = info.num_cores, info.num_subcores
    nw = nc * ns
    n_blocks = nb * ni
    assert n_blocks % nw == 0
    blocks_per_w = n_blocks // nw
    mesh = plsc.VectorSubcoreMesh(core_axis_name="c", subcore_axis_name="s")

    @functools.partial(
        pl.kernel,
        mesh=mesh,
        compiler_params=pltpu.CompilerParams(needs_layout_passes=False),
        out_type=[
            jax.ShapeDtypeStruct((nb, ni, D_MODEL, NJ), jnp.float32),
            jax.ShapeDtypeStruct((nb, ni, D_MODEL, NJ), jnp.float32),
        ],
        scratch_types=[
            pltpu.VMEM((2 * D_MODEL * VOCAB,), jnp.float32),
            pltpu.VMEM((NJ,), jnp.int32),
            pltpu.VMEM((D_MODEL, NJ), jnp.float32),
            pltpu.VMEM((D_MODEL, NJ), jnp.float32),
            pltpu.SemaphoreType.DMA,
            pltpu.SemaphoreType.DMA,
        ],
    )
    def gather_kernel(idx_hbm, tab_hbm, outk_hbm, outv_hbm,
                      tab_v, idx_v, kblk, vblk, semk, semv):
        wid = lax.axis_index("s") * nc + lax.axis_index("c")
        pltpu.sync_copy(tab_hbm, tab_v)

        def block_body(t, carry):
            blk = wid * blocks_per_w + t
            b = blk // ni
            i = blk % ni
            pltpu.sync_copy(idx_hbm.at[b, i], idx_v)

            for co in CHUNK_OFFS:
                idx16 = idx_v[pl.ds(co, LANES)]

                def d_body(d, kidx):
                    kvals = plsc.load_gather(tab_v, [kidx])
                    vvals = plsc.load_gather(tab_v, [kidx + D_MODEL * VOCAB])
                    kblk[d, pl.ds(co, LANES)] = kvals
                    vblk[d, pl.ds(co, LANES)] = vvals
                    return kidx + VOCAB

                lax.fori_loop(0, D_MODEL, d_body, idx16, unroll=8)

            ck = pltpu.async_copy(kblk, outk_hbm.at[b, i], semk)
            cv = pltpu.async_copy(vblk, outv_hbm.at[b, i], semv)
            ck.wait()
            cv.wait()
            return carry

        lax.fori_loop(0, blocks_per_w, block_body, 0)

    return gather_kernel


def kernel(inputs, relation_type, parent_table, brother_table):
    table = parent_table if relation_type == "parent" else brother_table
    nb, ni = inputs.shape[0], inputs.shape[1]
    idx = inputs.astype(jnp.int32)
    tab_t = table.T.reshape(-1)
    outk, outv = _make_gather(nb, ni)(idx, tab_t)
    return (outk.transpose(0, 1, 3, 2), outv.transpose(0, 1, 3, 2))


# speedup vs baseline: 5.7480x; 1.3057x over previous
import functools

import jax
import jax.numpy as jnp
from jax import lax
from jax.experimental import pallas as pl
from jax.experimental.pallas import tpu as pltpu
from jax.experimental.pallas import tpu_sc as plsc

D_MODEL = 64
VOCAB = 66
NJ = 200
LANES = 16
CHUNK_OFFS = tuple(range(0, NJ - LANES + 1, LANES)) + (NJ - LANES,)


@functools.lru_cache(maxsize=None)
def _make_gather(nb: int, ni: int):
    info = plsc.get_sparse_core_info()
    nc, ns = info.num_cores, info.num_subcores
    nw = nc * ns
    n_blocks = nb * ni
    assert n_blocks % nw == 0
    blocks_per_w = n_blocks // nw
    mesh = plsc.VectorSubcoreMesh(core_axis_name="c", subcore_axis_name="s")

    @functools.partial(
        pl.kernel,
        mesh=mesh,
        compiler_params=pltpu.CompilerParams(needs_layout_passes=False),
        out_type=[
            jax.ShapeDtypeStruct((nb, ni, D_MODEL, NJ), jnp.float32),
            jax.ShapeDtypeStruct((nb, ni, D_MODEL, NJ), jnp.float32),
        ],
        scratch_types=[
            pltpu.VMEM((2 * D_MODEL * VOCAB,), jnp.float32),
            pltpu.VMEM((2, NJ), jnp.int32),
            pltpu.VMEM((2 * D_MODEL, NJ), jnp.float32),
            pltpu.VMEM((2 * D_MODEL, NJ), jnp.float32),
            pltpu.SemaphoreType.DMA((2,)),
            pltpu.SemaphoreType.DMA((2,)),
            pltpu.SemaphoreType.DMA((2,)),
        ],
    )
    def gather_kernel(idx_hbm, tab_hbm, outk_hbm, outv_hbm,
                      tab_v, idx_v, kblk, vblk, semi, semk, semv):
        wid = lax.axis_index("s") * nc + lax.axis_index("c")
        base = wid * blocks_per_w
        pltpu.sync_copy(tab_hbm, tab_v)

        def bi(blk):
            return blk // ni, blk % ni

        def start_idx(blk, u):
            b, i = bi(blk)
            pltpu.async_copy(idx_hbm.at[b, i], idx_v.at[u], semi.at[u])

        start_idx(base, 0)
        start_idx(base + 1, 1)

        def block_body(t, carry):
            u = t % 2
            blk = base + t
            b, i = bi(blk)
            kb = kblk.at[pl.ds(u * D_MODEL, D_MODEL)]
            vb = vblk.at[pl.ds(u * D_MODEL, D_MODEL)]

            @pl.when(t >= 2)
            def _():
                pltpu.make_async_copy(kb, outk_hbm.at[b, i], semk.at[u]).wait()
                pltpu.make_async_copy(vb, outv_hbm.at[b, i], semv.at[u]).wait()

            pltpu.make_async_copy(idx_hbm.at[b, i], idx_v.at[u], semi.at[u]).wait()

            for co in CHUNK_OFFS:
                idx16 = idx_v[u, pl.ds(co, LANES)]

                def d_body(d, kidx):
                    kvals = plsc.load_gather(tab_v, [kidx])
                    vvals = plsc.load_gather(tab_v, [kidx + D_MODEL * VOCAB])
                    kblk[u * D_MODEL + d, pl.ds(co, LANES)] = kvals
                    vblk[u * D_MODEL + d, pl.ds(co, LANES)] = vvals
                    return kidx + VOCAB

                lax.fori_loop(0, D_MODEL, d_body, idx16, unroll=8)

            @pl.when(t + 2 < blocks_per_w)
            def _():
                start_idx(blk + 2, u)

            pltpu.async_copy(kb, outk_hbm.at[b, i], semk.at[u])
            pltpu.async_copy(vb, outv_hbm.at[b, i], semv.at[u])
            return carry

        lax.fori_loop(0, blocks_per_w, block_body, 0)

        for t in (blocks_per_w - 2, blocks_per_w - 1):
            u = t % 2
            b, i = bi(base + t)
            kb = kblk.at[pl.ds(u * D_MODEL, D_MODEL)]
            vb = vblk.at[pl.ds(u * D_MODEL, D_MODEL)]
            pltpu.make_async_copy(kb, outk_hbm.at[b, i], semk.at[u]).wait()
            pltpu.make_async_copy(vb, outv_hbm.at[b, i], semv.at[u]).wait()

    return gather_kernel


def kernel(inputs, relation_type, parent_table, brother_table):
    table = parent_table if relation_type == "parent" else brother_table
    nb, ni = inputs.shape[0], inputs.shape[1]
    idx = inputs.astype(jnp.int32)
    tab_t = table.T.reshape(-1)
    outk, outv = _make_gather(nb, ni)(idx, tab_t)
    return (outk.transpose(0, 1, 3, 2), outv.transpose(0, 1, 3, 2))


# speedup vs baseline: 10.5272x; 1.8315x over previous
import functools

import jax
import jax.numpy as jnp
from jax import lax
from jax.experimental import pallas as pl
from jax.experimental.pallas import tpu as pltpu
from jax.experimental.pallas import tpu_sc as plsc

D_MODEL = 64
VOCAB = 66
NJ = 200
LANES = 16
CHUNK_OFFS = tuple(range(0, NJ - LANES + 1, LANES)) + (NJ - LANES,)


@functools.lru_cache(maxsize=None)
def _make_gather(nb: int, ni: int):
    info = plsc.get_sparse_core_info()
    nc, ns = info.num_cores, info.num_subcores
    nw = nc * ns
    n_blocks = nb * ni
    assert n_blocks % nw == 0
    blocks_per_w = n_blocks // nw
    mesh = plsc.VectorSubcoreMesh(core_axis_name="c", subcore_axis_name="s")

    @functools.partial(
        pl.kernel,
        mesh=mesh,
        compiler_params=pltpu.CompilerParams(needs_layout_passes=False),
        out_type=[
            jax.ShapeDtypeStruct((nb, ni, D_MODEL, NJ), jnp.float32),
            jax.ShapeDtypeStruct((nb, ni, D_MODEL, NJ), jnp.float32),
        ],
        scratch_types=[
            pltpu.VMEM((2 * D_MODEL * VOCAB,), jnp.float32),
            pltpu.VMEM((2, NJ), jnp.int32),
            pltpu.VMEM((2 * D_MODEL, NJ), jnp.float32),
            pltpu.VMEM((2 * D_MODEL, NJ), jnp.float32),
            pltpu.SemaphoreType.DMA((2,)),
            pltpu.SemaphoreType.DMA((2,)),
            pltpu.SemaphoreType.DMA((2,)),
        ],
    )
    def gather_kernel(idx_hbm, tab_hbm, outk_hbm, outv_hbm,
                      tab_v, idx_v, kblk, vblk, semi, semk, semv):
        wid = lax.axis_index("s") * nc + lax.axis_index("c")
        base = wid * blocks_per_w
        pltpu.sync_copy(tab_hbm, tab_v)

        def bi(blk):
            return blk // ni, blk % ni

        def start_idx(blk, u):
            b, i = bi(blk)
            pltpu.async_copy(idx_hbm.at[b, i], idx_v.at[u], semi.at[u])

        start_idx(base, 0)
        start_idx(base + 1, 1)

        def block_body(t, carry):
            u = t % 2
            blk = base + t
            b, i = bi(blk)
            kb = kblk.at[pl.ds(u * D_MODEL, D_MODEL)]
            vb = vblk.at[pl.ds(u * D_MODEL, D_MODEL)]

            @pl.when(t >= 2)
            def _():
                pltpu.make_async_copy(kb, outk_hbm.at[b, i], semk.at[u]).wait()
                pltpu.make_async_copy(vb, outv_hbm.at[b, i], semv.at[u]).wait()

            pltpu.make_async_copy(idx_hbm.at[b, i], idx_v.at[u], semi.at[u]).wait()

            idxs = tuple(idx_v[u, pl.ds(co, LANES)] for co in CHUNK_OFFS)
            addrs0 = []
            targets = []
            for ci, co in enumerate(CHUNK_OFFS):
                addrs0.append(idxs[ci])
                targets.append((kblk, co))
                addrs0.append(idxs[ci] + D_MODEL * VOCAB)
                targets.append((vblk, co))
            n_g = len(addrs0)
            depth = 4

            def d_body(d, addrs):
                row = u * D_MODEL + d
                vals = [None] * n_g
                for g in range(n_g):
                    vals[g] = plsc.load_gather(tab_v, [addrs[g]])
                    if g >= depth:
                        ref, co = targets[g - depth]
                        ref[row, pl.ds(co, LANES)] = vals[g - depth]
                for g in range(n_g - depth, n_g):
                    ref, co = targets[g]
                    ref[row, pl.ds(co, LANES)] = vals[g]
                return tuple(a + VOCAB for a in addrs)

            lax.fori_loop(0, D_MODEL, d_body, tuple(addrs0), unroll=1)

            @pl.when(t + 2 < blocks_per_w)
            def _():
                start_idx(blk + 2, u)

            pltpu.async_copy(kb, outk_hbm.at[b, i], semk.at[u])
            pltpu.async_copy(vb, outv_hbm.at[b, i], semv.at[u])
            return carry

        lax.fori_loop(0, blocks_per_w, block_body, 0)

        for t in (blocks_per_w - 2, blocks_per_w - 1):
            u = t % 2
            b, i = bi(base + t)
            kb = kblk.at[pl.ds(u * D_MODEL, D_MODEL)]
            vb = vblk.at[pl.ds(u * D_MODEL, D_MODEL)]
            pltpu.make_async_copy(kb, outk_hbm.at[b, i], semk.at[u]).wait()
            pltpu.make_async_copy(vb, outv_hbm.at[b, i], semv.at[u]).wait()

    return gather_kernel


def kernel(inputs, relation_type, parent_table, brother_table):
    table = parent_table if relation_type == "parent" else brother_table
    nb, ni = inputs.shape[0], inputs.shape[1]
    idx = inputs.astype(jnp.int32)
    tab_t = table.T.reshape(-1)
    outk, outv = _make_gather(nb, ni)(idx, tab_t)
    return (outk.transpose(0, 1, 3, 2), outv.transpose(0, 1, 3, 2))


# speedup vs baseline: 10.9319x; 1.0384x over previous
import functools

import jax
import jax.numpy as jnp
from jax import lax
from jax.experimental import pallas as pl
from jax.experimental.pallas import tpu as pltpu
from jax.experimental.pallas import tpu_sc as plsc

D_MODEL = 64
VOCAB = 66
NJ = 200
LANES = 16
CHUNK_OFFS = tuple(range(0, NJ - LANES + 1, LANES)) + (NJ - LANES,)


@functools.lru_cache(maxsize=None)
def _make_gather(nb: int, ni: int):
    info = plsc.get_sparse_core_info()
    nc, ns = info.num_cores, info.num_subcores
    nw = nc * ns
    n_blocks = nb * ni
    assert n_blocks % nw == 0
    blocks_per_w = n_blocks // nw
    mesh = plsc.VectorSubcoreMesh(core_axis_name="c", subcore_axis_name="s")

    @functools.partial(
        pl.kernel,
        mesh=mesh,
        compiler_params=pltpu.CompilerParams(needs_layout_passes=False),
        out_type=[
            jax.ShapeDtypeStruct((nb, ni, D_MODEL, NJ), jnp.float32),
            jax.ShapeDtypeStruct((nb, ni, D_MODEL, NJ), jnp.float32),
        ],
        scratch_types=[
            pltpu.VMEM((2 * D_MODEL * VOCAB,), jnp.float32),
            pltpu.VMEM((2, NJ), jnp.int32),
            pltpu.VMEM((2 * D_MODEL, NJ), jnp.float32),
            pltpu.VMEM((2 * D_MODEL, NJ), jnp.float32),
            pltpu.SemaphoreType.DMA((2,)),
            pltpu.SemaphoreType.DMA((2,)),
            pltpu.SemaphoreType.DMA((2,)),
        ],
    )
    def gather_kernel(idx_hbm, tab_hbm, outk_hbm, outv_hbm,
                      tab_v, idx_v, kblk, vblk, semi, semk, semv):
        wid = lax.axis_index("s") * nc + lax.axis_index("c")
        base = wid * blocks_per_w
        pltpu.sync_copy(tab_hbm, tab_v)

        def bi(blk):
            return blk // ni, blk % ni

        def start_idx(blk, u):
            b, i = bi(blk)
            pltpu.async_copy(idx_hbm.at[b, i], idx_v.at[u], semi.at[u])

        start_idx(base, 0)
        start_idx(base + 1, 1)

        def block_body(t, carry):
            u = t % 2
            blk = base + t
            b, i = bi(blk)
            kb = kblk.at[pl.ds(u * D_MODEL, D_MODEL)]
            vb = vblk.at[pl.ds(u * D_MODEL, D_MODEL)]

            @pl.when(t >= 2)
            def _():
                pltpu.make_async_copy(kb, outk_hbm.at[b, i], semk.at[u]).wait()
                pltpu.make_async_copy(vb, outv_hbm.at[b, i], semv.at[u]).wait()

            pltpu.make_async_copy(idx_hbm.at[b, i], idx_v.at[u], semi.at[u]).wait()

            idxs = tuple(idx_v[u, pl.ds(co, LANES)] for co in CHUNK_OFFS)
            addrs0 = []
            targets = []
            for ci, co in enumerate(CHUNK_OFFS):
                addrs0.append(idxs[ci])
                targets.append((kblk, co))
                addrs0.append(idxs[ci] + D_MODEL * VOCAB)
                targets.append((vblk, co))
            n_g = len(addrs0)
            depth = 4

            def d_body(d, addrs):
                row = u * D_MODEL + d
                vals = [None] * n_g
                for g in range(n_g):
                    vals[g] = plsc.load_gather(tab_v, [addrs[g]])
                    if g >= depth:
                        ref, co = targets[g - depth]
                        ref[row, pl.ds(co, LANES)] = vals[g - depth]
                for g in range(n_g - depth, n_g):
                    ref, co = targets[g]
                    ref[row, pl.ds(co, LANES)] = vals[g]
                return tuple(a + VOCAB for a in addrs)

            lax.fori_loop(0, D_MODEL, d_body, tuple(addrs0), unroll=2)

            @pl.when(t + 2 < blocks_per_w)
            def _():
                start_idx(blk + 2, u)

            pltpu.async_copy(kb, outk_hbm.at[b, i], semk.at[u])
            pltpu.async_copy(vb, outv_hbm.at[b, i], semv.at[u])
            return carry

        lax.fori_loop(0, blocks_per_w, block_body, 0)

        for t in (blocks_per_w - 2, blocks_per_w - 1):
            u = t % 2
            b, i = bi(base + t)
            kb = kblk.at[pl.ds(u * D_MODEL, D_MODEL)]
            vb = vblk.at[pl.ds(u * D_MODEL, D_MODEL)]
            pltpu.make_async_copy(kb, outk_hbm.at[b, i], semk.at[u]).wait()
            pltpu.make_async_copy(vb, outv_hbm.at[b, i], semv.at[u]).wait()

    return gather_kernel


def kernel(inputs, relation_type, parent_table, brother_table):
    table = parent_table if relation_type == "parent" else brother_table
    nb, ni = inputs.shape[0], inputs.shape[1]
    idx = inputs.astype(jnp.int32)
    tab_t = table.T.reshape(-1)
    outk, outv = _make_gather(nb, ni)(idx, tab_t)
    return (outk.transpose(0, 1, 3, 2), outv.transpose(0, 1, 3, 2))
